# Initial kernel scaffold; baseline (speedup 1.0000x reference)
import jax
import jax.numpy as jnp
from jax.experimental import pallas as pl


def kernel(x, W1, b1, W2, b2):
    raise NotImplementedError("write your pallas kernel here")


# speedup vs baseline: 1.9440x; 1.9440x over previous
import functools

import jax
import jax.numpy as jnp
from jax.experimental import pallas as pl

_TOKENS = 32768
_D_IN = 768
_D_HID = 256
_N_EXPERTS = 64
_BLOCK = 1024


def _gating_body(x_ref, w1_ref, b1_ref, w2_ref, b2_ref, idx_ref, gate_ref):
    h = jnp.dot(x_ref[...], w1_ref[...], preferred_element_type=jnp.float32)
    h = jnp.maximum(h + b1_ref[...], 0.0)
    logits = jnp.dot(h, w2_ref[...], preferred_element_type=jnp.float32)
    logits = logits + b2_ref[...]

    iota = jax.lax.broadcasted_iota(jnp.int32, logits.shape, 1)
    m1 = jnp.max(logits, axis=1, keepdims=True)
    i1 = jnp.min(jnp.where(logits == m1, iota, _N_EXPERTS), axis=1,
                 keepdims=True)
    masked = jnp.where(iota == i1, -jnp.inf, logits)
    m2 = jnp.max(masked, axis=1, keepdims=True)
    i2 = jnp.min(jnp.where(masked == m2, iota, _N_EXPERTS), axis=1,
                 keepdims=True)

    e = jnp.exp(m2 - m1)
    denom = 1.0 + e
    g1 = 1.0 / denom
    g2 = e / denom

    idx_ref[...] = jnp.concatenate([i1, i2], axis=1)
    gate_ref[...] = jnp.concatenate([g1, g2], axis=1)


@functools.partial(jax.jit, static_argnames=("interpret",))
def kernel(x, W1, b1, W2, b2, interpret=False):
    b1r = b1.reshape(1, _D_HID)
    b2r = b2.reshape(1, _N_EXPERTS)
    grid = (_TOKENS // _BLOCK,)
    idx, gates = pl.pallas_call(
        _gating_body,
        grid=grid,
        in_specs=[
            pl.BlockSpec((_BLOCK, _D_IN), lambda i: (i, 0)),
            pl.BlockSpec((_D_IN, _D_HID), lambda i: (0, 0)),
            pl.BlockSpec((1, _D_HID), lambda i: (0, 0)),
            pl.BlockSpec((_D_HID, _N_EXPERTS), lambda i: (0, 0)),
            pl.BlockSpec((1, _N_EXPERTS), lambda i: (0, 0)),
        ],
        out_specs=[
            pl.BlockSpec((_BLOCK, 2), lambda i: (i, 0)),
            pl.BlockSpec((_BLOCK, 2), lambda i: (i, 0)),
        ],
        out_shape=[
            jax.ShapeDtypeStruct((_TOKENS, 2), jnp.int32),
            jax.ShapeDtypeStruct((_TOKENS, 2), jnp.float32),
        ],
        interpret=interpret,
    )(x, W1, b1r, W2, b2r)
    return idx, gates


# speedup vs baseline: 2.0338x; 1.0462x over previous
import functools

import jax
import jax.numpy as jnp
from jax.experimental import pallas as pl

_TOKENS = 32768
_D_IN = 768
_D_HID = 256
_N_EXPERTS = 64
_BLOCK = 1024


def _gating_body(x_ref, w1_ref, b1_ref, w2_ref, b2_ref, idx_ref, gate_ref):
    h = jnp.dot(x_ref[...], w1_ref[...], preferred_element_type=jnp.float32)
    h = jnp.maximum(h + b1_ref[...], 0.0)
    logits = jnp.dot(h, w2_ref[...], preferred_element_type=jnp.float32)
    logits = logits + b2_ref[...]

    neg_iota = -jax.lax.broadcasted_iota(
        jnp.int32, logits.shape, 1).astype(jnp.float32)
    ninf = jnp.float32(-jnp.inf)
    m1 = jnp.max(logits, axis=1, keepdims=True)
    ni1 = jnp.max(jnp.where(logits == m1, neg_iota, ninf), axis=1,
                  keepdims=True)
    masked = jnp.where(neg_iota == ni1, ninf, logits)
    m2 = jnp.max(masked, axis=1, keepdims=True)
    ni2 = jnp.max(jnp.where(masked == m2, neg_iota, ninf), axis=1,
                  keepdims=True)
    i1 = (-ni1).astype(jnp.int32)
    i2 = (-ni2).astype(jnp.int32)

    e = jnp.exp(m2 - m1)
    denom = 1.0 + e
    g1 = 1.0 / denom
    g2 = e / denom

    idx_ref[...] = jnp.concatenate([i1, i2], axis=1)
    gate_ref[...] = jnp.concatenate([g1, g2], axis=1)


@functools.partial(jax.jit, static_argnames=("interpret",))
def kernel(x, W1, b1, W2, b2, interpret=False):
    b1r = b1.reshape(1, _D_HID)
    b2r = b2.reshape(1, _N_EXPERTS)
    grid = (_TOKENS // _BLOCK,)
    idx, gates = pl.pallas_call(
        _gating_body,
        grid=grid,
        in_specs=[
            pl.BlockSpec((_BLOCK, _D_IN), lambda i: (i, 0)),
            pl.BlockSpec((_D_IN, _D_HID), lambda i: (0, 0)),
            pl.BlockSpec((1, _D_HID), lambda i: (0, 0)),
            pl.BlockSpec((_D_HID, _N_EXPERTS), lambda i: (0, 0)),
            pl.BlockSpec((1, _N_EXPERTS), lambda i: (0, 0)),
        ],
        out_specs=[
            pl.BlockSpec((_BLOCK, 2), lambda i: (i, 0)),
            pl.BlockSpec((_BLOCK, 2), lambda i: (i, 0)),
        ],
        out_shape=[
            jax.ShapeDtypeStruct((_TOKENS, 2), jnp.int32),
            jax.ShapeDtypeStruct((_TOKENS, 2), jnp.float32),
        ],
        interpret=interpret,
    )(x, W1, b1r, W2, b2r)
    return idx, gates


# speedup vs baseline: 2.3853x; 1.1728x over previous
import functools

import jax
import jax.numpy as jnp
from jax.experimental import pallas as pl

_TOKENS = 32768
_D_IN = 768
_D_HID = 256
_N_EXPERTS = 64
_BLOCK = 2048


def _gating_body(x_ref, w1_ref, b1_ref, w2_ref, b2_ref, idx_ref, gate_ref):
    h = jnp.dot(x_ref[...], w1_ref[...], preferred_element_type=jnp.float32)
    h = jnp.maximum(h + b1_ref[...], 0.0)
    logits = jnp.dot(h, w2_ref[...], preferred_element_type=jnp.float32)
    logits = logits + b2_ref[...]

    neg_iota = -jax.lax.broadcasted_iota(
        jnp.int32, logits.shape, 1).astype(jnp.float32)
    ninf = jnp.float32(-jnp.inf)
    m1 = jnp.max(logits, axis=1, keepdims=True)
    ni1 = jnp.max(jnp.where(logits == m1, neg_iota, ninf), axis=1,
                  keepdims=True)
    masked = jnp.where(neg_iota == ni1, ninf, logits)
    m2 = jnp.max(masked, axis=1, keepdims=True)
    ni2 = jnp.max(jnp.where(masked == m2, neg_iota, ninf), axis=1,
                  keepdims=True)
    i1 = (-ni1).astype(jnp.int32)
    i2 = (-ni2).astype(jnp.int32)

    e = jnp.exp(m2 - m1)
    denom = 1.0 + e
    g1 = 1.0 / denom
    g2 = e / denom

    idx_ref[...] = jnp.concatenate([i1, i2], axis=1)
    gate_ref[...] = jnp.concatenate([g1, g2], axis=1)


@functools.partial(jax.jit, static_argnames=("interpret",))
def kernel(x, W1, b1, W2, b2, interpret=False):
    b1r = b1.reshape(1, _D_HID)
    b2r = b2.reshape(1, _N_EXPERTS)
    grid = (_TOKENS // _BLOCK,)
    idx, gates = pl.pallas_call(
        _gating_body,
        grid=grid,
        in_specs=[
            pl.BlockSpec((_BLOCK, _D_IN), lambda i: (i, 0)),
            pl.BlockSpec((_D_IN, _D_HID), lambda i: (0, 0)),
            pl.BlockSpec((1, _D_HID), lambda i: (0, 0)),
            pl.BlockSpec((_D_HID, _N_EXPERTS), lambda i: (0, 0)),
            pl.BlockSpec((1, _N_EXPERTS), lambda i: (0, 0)),
        ],
        out_specs=[
            pl.BlockSpec((_BLOCK, 2), lambda i: (i, 0)),
            pl.BlockSpec((_BLOCK, 2), lambda i: (i, 0)),
        ],
        out_shape=[
            jax.ShapeDtypeStruct((_TOKENS, 2), jnp.int32),
            jax.ShapeDtypeStruct((_TOKENS, 2), jnp.float32),
        ],
        interpret=interpret,
    )(x, W1, b1r, W2, b2r)
    return idx, gates


# speedup vs baseline: 2.4839x; 1.0414x over previous
import functools

import jax
import jax.numpy as jnp
from jax.experimental import pallas as pl

_TOKENS = 32768
_D_IN = 768
_D_HID = 256
_N_EXPERTS = 64
_BLOCK = 4096


def _gating_body(x_ref, w1_ref, b1_ref, w2_ref, b2_ref, idx_ref, gate_ref):
    h = jnp.dot(x_ref[...], w1_ref[...], preferred_element_type=jnp.float32)
    h = jnp.maximum(h + b1_ref[...], 0.0)
    logits = jnp.dot(h, w2_ref[...], preferred_element_type=jnp.float32)
    logits = logits + b2_ref[...]

    neg_iota = -jax.lax.broadcasted_iota(
        jnp.int32, logits.shape, 1).astype(jnp.float32)
    ninf = jnp.float32(-jnp.inf)
    m1 = jnp.max(logits, axis=1, keepdims=True)
    ni1 = jnp.max(jnp.where(logits == m1, neg_iota, ninf), axis=1,
                  keepdims=True)
    masked = jnp.where(neg_iota == ni1, ninf, logits)
    m2 = jnp.max(masked, axis=1, keepdims=True)
    ni2 = jnp.max(jnp.where(masked == m2, neg_iota, ninf), axis=1,
                  keepdims=True)
    i1 = (-ni1).astype(jnp.int32)
    i2 = (-ni2).astype(jnp.int32)

    e = jnp.exp(m2 - m1)
    denom = 1.0 + e
    g1 = 1.0 / denom
    g2 = e / denom

    idx_ref[...] = jnp.concatenate([i1, i2], axis=1)
    gate_ref[...] = jnp.concatenate([g1, g2], axis=1)


@functools.partial(jax.jit, static_argnames=("interpret",))
def kernel(x, W1, b1, W2, b2, interpret=False):
    b1r = b1.reshape(1, _D_HID)
    b2r = b2.reshape(1, _N_EXPERTS)
    grid = (_TOKENS // _BLOCK,)
    idx, gates = pl.pallas_call(
        _gating_body,
        grid=grid,
        in_specs=[
            pl.BlockSpec((_BLOCK, _D_IN), lambda i: (i, 0)),
            pl.BlockSpec((_D_IN, _D_HID), lambda i: (0, 0)),
            pl.BlockSpec((1, _D_HID), lambda i: (0, 0)),
            pl.BlockSpec((_D_HID, _N_EXPERTS), lambda i: (0, 0)),
            pl.BlockSpec((1, _N_EXPERTS), lambda i: (0, 0)),
        ],
        out_specs=[
            pl.BlockSpec((_BLOCK, 2), lambda i: (i, 0)),
            pl.BlockSpec((_BLOCK, 2), lambda i: (i, 0)),
        ],
        out_shape=[
            jax.ShapeDtypeStruct((_TOKENS, 2), jnp.int32),
            jax.ShapeDtypeStruct((_TOKENS, 2), jnp.float32),
        ],
        interpret=interpret,
    )(x, W1, b1r, W2, b2r)
    return idx, gates
